# baseline (speedup 1.0000x reference)
import functools

import jax
import jax.numpy as jnp
from jax import lax
from jax.experimental import pallas as pl
from jax.experimental.pallas import tpu as pltpu
from jax.experimental.pallas import tpu_sc as plsc

_NUM_CORES = 2
_NUM_SUBCORES = 16
_NUM_WORKERS = _NUM_CORES * _NUM_SUBCORES

_B = 50000
_D = 128
_S1 = 11
_C = 104
_NCHUNK = 512
_CPW = _NCHUNK // _NUM_WORKERS
_BPW = _CPW * _C
_BPAD = 50048
_INV = 1.0 / _S1
_NBUF = 4


def _sc_body(feat_hbm, idxt_hbm, out_hbm, idx_v, acc, obuf,
             gsem0, gsem1, gsem2, gsem3, osem0, osem1):
    wid = lax.axis_index("c") * _NUM_SUBCORES + lax.axis_index("s")
    gsems = (gsem0, gsem1, gsem2, gsem3)
    osems = (osem0, osem1)
    zeros = jnp.zeros((16,), jnp.float32)

    base = jnp.minimum(wid * _BPW, _B - _BPW)
    for k in range(_S1):
        pltpu.sync_copy(idxt_hbm.at[pl.ds(k * _BPAD + base, _BPW)],
                        idx_v.at[pl.ds(k * _BPW, _BPW)])

    def chunk_off(t):
        row0 = jnp.minimum((wid * _CPW + t) * _C, _B - _C)
        return row0, row0 - base

    def zero(b):
        def zrow(r, _):
            for j in range(_D // 16):
                acc[b, r, pl.ds(j * 16, 16)] = zeros
            return _
        lax.fori_loop(0, _C, zrow, None)

    def fire(t, b):
        _, off = chunk_off(t)
        for k in range(_S1):
            idx = idx_v.at[pl.ds(k * _BPW + off, _C)]
            pltpu.async_copy(feat_hbm.at[idx], acc.at[b], gsems[b], add=True)

    def drain(b):
        for k in range(_S1):
            pltpu.make_async_copy(feat_hbm.at[pl.ds(0, _C)], acc.at[b],
                                  gsems[b]).wait()

    def scale_zero(b, p):
        def srow(r, _):
            for j in range(_D // 16):
                sl = pl.ds(j * 16, 16)
                obuf[p, r, sl] = acc[b, r, sl] * _INV
                acc[b, r, sl] = zeros
            return _
        lax.fori_loop(0, _C, srow, None)

    def out_wait(p):
        pltpu.make_async_copy(obuf.at[p], out_hbm.at[pl.ds(0, _C)],
                              osems[p]).wait()

    def out_start(t, p):
        row0, _ = chunk_off(t)
        pltpu.async_copy(obuf.at[p], out_hbm.at[pl.ds(row0, _C)], osems[p])

    for b in range(_NBUF):
        zero(b)
    for t0 in range(_NBUF - 1):
        fire(t0, t0)

    def quad_body(t4, _):
        for j in range(_NBUF):
            t = _NBUF * t4 + j
            drain(j)

            @pl.when(t >= 2)
            def _():
                out_wait(j % 2)

            scale_zero(j, j % 2)

            @pl.when(t + _NBUF - 1 < _CPW)
            def _():
                fire(t + _NBUF - 1, (j + _NBUF - 1) % _NBUF)

            out_start(t, j % 2)
        return _

    lax.fori_loop(0, _CPW // _NBUF, quad_body, None)
    out_wait(0)
    out_wait(1)


@functools.partial(
    pl.kernel,
    out_type=jax.ShapeDtypeStruct((_B, _D), jnp.float32),
    mesh=plsc.VectorSubcoreMesh(
        core_axis_name="c", subcore_axis_name="s",
        num_cores=_NUM_CORES, num_subcores=_NUM_SUBCORES,
    ),
    scratch_types=[
        pltpu.VMEM((_S1 * _BPW,), jnp.int32),
        pltpu.VMEM((_NBUF, _C, _D), jnp.float32),
        pltpu.VMEM((2, _C, _D), jnp.float32),
        pltpu.SemaphoreType.DMA,
        pltpu.SemaphoreType.DMA,
        pltpu.SemaphoreType.DMA,
        pltpu.SemaphoreType.DMA,
        pltpu.SemaphoreType.DMA,
        pltpu.SemaphoreType.DMA,
    ],
)
def _mean_agg_sc(feat_hbm, idxt_hbm, out_hbm, idx_v, acc, obuf,
                 gsem0, gsem1, gsem2, gsem3, osem0, osem1):
    _sc_body(feat_hbm, idxt_hbm, out_hbm, idx_v, acc, obuf,
             gsem0, gsem1, gsem2, gsem3, osem0, osem1)


def kernel(features, nodes, neighbours_full, num_sample):
    s = neighbours_full.shape[1]
    idxt = jnp.concatenate([neighbours_full.T, nodes[None, :]], axis=0)
    idxt = idxt + (num_sample - s)
    idxt = jnp.pad(idxt, ((0, 0), (0, _BPAD - _B))).reshape(-1)
    return _mean_agg_sc(features, idxt)
